# baseline (speedup 1.0000x reference)
import functools

import jax
import jax.numpy as jnp
from jax import lax
from jax.experimental import pallas as pl
from jax.experimental.pallas import tpu as pltpu
from jax.experimental.pallas import tpu_sc as plsc

B, L, D = 1024, 200, 128
N = B * L
NC, NS = 2, 16
NW = NC * NS
RPW = N // NW
C = 128
NCHUNK = RPW // C

_mesh = plsc.VectorSubcoreMesh(core_axis_name="c", subcore_axis_name="s")


def _sc_body(x_hbm, yidx_hbm, table_hbm, out_hbm, idx_v, emb_v, sem):
    wid = lax.axis_index("s") * NC + lax.axis_index("c")
    base0 = wid * RPW

    @pl.loop(0, NCHUNK)
    def _(g):
        base = base0 + g * C
        pltpu.sync_copy(yidx_hbm.at[pl.ds(base, C)], idx_v)
        pltpu.async_copy(table_hbm.at[idx_v], emb_v, sem).wait()
        pltpu.sync_copy(x_hbm.at[pl.ds(base, C)],
                        out_hbm.at[pl.ds(base, C), pl.ds(0, D)])
        pltpu.sync_copy(emb_v, out_hbm.at[pl.ds(base, C), pl.ds(D, D)])


@jax.jit
def kernel(x, labels_pointer, emb_table):
    xf = x.reshape(N, D)
    yidx = jnp.repeat(labels_pointer, L)
    call = pl.kernel(
        _sc_body,
        out_type=jax.ShapeDtypeStruct((N, 2 * D), x.dtype),
        mesh=_mesh,
        scratch_types=[
            pltpu.VMEM((C,), jnp.int32),
            pltpu.VMEM((C, D), jnp.float32),
            pltpu.SemaphoreType.DMA,
        ],
    )
    out = call(xf, yidx, emb_table)
    return out.reshape(B, L, 2 * D)


# speedup vs baseline: 1.0027x; 1.0027x over previous
import functools

import jax
import jax.numpy as jnp
from jax import lax
from jax.experimental import pallas as pl
from jax.experimental.pallas import tpu as pltpu
from jax.experimental.pallas import tpu_sc as plsc

B, L, D = 1024, 200, 128
N = B * L
NC, NS = 2, 16
NW = NC * NS
RPW = N // NW
C = 128
NCHUNK = RPW // C
NBUF = 5

_mesh = plsc.VectorSubcoreMesh(core_axis_name="c", subcore_axis_name="s")


def _sc_body(x_hbm, yidx_hbm, table_hbm, out_hbm, idx_v, emb_v,
             xsem, gsem, wsem):
    wid = lax.axis_index("s") * NC + lax.axis_index("c")
    base0 = wid * RPW

    xcopy = pltpu.async_copy(
        x_hbm.at[pl.ds(base0, RPW)],
        out_hbm.at[pl.ds(base0, RPW), pl.ds(0, D)], xsem)

    pltpu.sync_copy(yidx_hbm.at[pl.ds(base0, RPW)], idx_v)

    def gather(g, b):
        return pltpu.make_async_copy(table_hbm.at[idx_v.at[pl.ds(g * C, C)]],
                                     emb_v.at[b], gsem.at[b])

    def drain(g, b):
        return pltpu.make_async_copy(
            emb_v.at[b],
            out_hbm.at[pl.ds(base0 + g * C, C), pl.ds(D, D)], wsem.at[b])

    for b in range(NBUF):
        gather(b, b).start()

    @pl.loop(0, NCHUNK - NBUF, step=NBUF)
    def _(g0):
        for b in range(NBUF):
            gather(g0 + b, b).wait()
            drain(g0 + b, b).start()
        for b in range(NBUF):
            drain(g0 + b, b).wait()
            gather(g0 + b + NBUF, b).start()

    for b in range(NBUF):
        g = NCHUNK - NBUF + b
        gather(g, b).wait()
        drain(g, b).start()
        drain(g, b).wait()

    xcopy.wait()


@jax.jit
def kernel(x, labels_pointer, emb_table):
    xf = x.reshape(N, D)
    yidx = jnp.repeat(labels_pointer, L)
    call = pl.kernel(
        _sc_body,
        out_type=jax.ShapeDtypeStruct((N, 2 * D), x.dtype),
        mesh=_mesh,
        scratch_types=[
            pltpu.VMEM((RPW,), jnp.int32),
            pltpu.VMEM((NBUF, C, D), jnp.float32),
            pltpu.SemaphoreType.DMA,
            pltpu.SemaphoreType.DMA((NBUF,)),
            pltpu.SemaphoreType.DMA((NBUF,)),
        ],
    )
    out = call(xf, yidx, emb_table)
    return out.reshape(B, L, 2 * D)
